# baseline (device time: 517791 ns/iter reference)
import jax
import jax.numpy as jnp
from jax import lax
from jax.experimental import pallas as pl
from jax.experimental.pallas import tpu as pltpu

N_DEV = 8
E_LOC = 8
E_TOT = N_DEV * E_LOC
C = 64
D = 1024


def _body(x_ref, xsend_ref, sw_ref, ew_ref, out_ref, yret_ref,
          xrecv_ref, we_ref, ybuf_ref,
          ssx, rsx, ssy, rsy, wsem, msem, ysem):
    me = lax.axis_index("i")

    barrier = pltpu.get_barrier_semaphore()
    for k in range(1, N_DEV):
        peer = lax.rem(me + k, N_DEV)
        pl.semaphore_signal(barrier, inc=1, device_id=(peer,),
                            device_id_type=pl.DeviceIdType.MESH)
    pl.semaphore_wait(barrier, N_DEV - 1)

    disp = []
    for k in range(1, N_DEV):
        peer = lax.rem(me + k, N_DEV)
        r = pltpu.make_async_remote_copy(
            src_ref=xsend_ref.at[k],
            dst_ref=xrecv_ref.at[k],
            send_sem=ssx.at[k],
            recv_sem=rsx.at[k],
            device_id=(peer,),
            device_id_type=pl.DeviceIdType.MESH,
        )
        r.start()
        disp.append(r)
    selfcp = pltpu.make_async_copy(xsend_ref.at[0], xrecv_ref.at[0], msem.at[0])
    selfcp.start()
    pltpu.make_async_copy(ew_ref.at[0], we_ref.at[0], wsem.at[0]).start()

    out_ref[...] = jnp.dot(x_ref[...], sw_ref[...],
                           preferred_element_type=jnp.float32)

    selfcp.wait()
    for r in disp:
        r.wait_recv()

    ret_rdmas = {0: [], 1: []}
    ret_local = {0: None, 1: None}
    for e in range(E_LOC):
        par = e % 2
        pltpu.make_async_copy(ew_ref.at[e], we_ref.at[par], wsem.at[par]).wait()
        if e + 1 < E_LOC:
            pltpu.make_async_copy(ew_ref.at[e + 1], we_ref.at[(e + 1) % 2],
                                  wsem.at[(e + 1) % 2]).start()
        if e >= 2:
            for r in ret_rdmas[par]:
                r.wait_send()
            ret_local[par].wait()
        a = xrecv_ref[:, e, :, :].reshape(N_DEV * C, D)
        y = jnp.dot(a, we_ref[par], preferred_element_type=jnp.float32)
        ybuf_ref[par] = y.reshape(N_DEV, C, D)
        lst = []
        for k in range(1, N_DEV):
            src_dev = lax.rem(me - k + N_DEV, N_DEV)
            r = pltpu.make_async_remote_copy(
                src_ref=ybuf_ref.at[par, k],
                dst_ref=yret_ref.at[k, e],
                send_sem=ssy.at[par, k],
                recv_sem=rsy.at[k, e],
                device_id=(src_dev,),
                device_id_type=pl.DeviceIdType.MESH,
            )
            r.start()
            lst.append(r)
        ret_rdmas[par] = lst
        lc = pltpu.make_async_copy(ybuf_ref.at[par, 0], yret_ref.at[0, e],
                                   ysem.at[par])
        lc.start()
        ret_local[par] = lc

    for par in (0, 1):
        for r in ret_rdmas[par]:
            r.wait_send()
        if ret_local[par] is not None:
            ret_local[par].wait()
    for r in disp:
        r.wait_send()
    for k in range(1, N_DEV):
        for e in range(E_LOC):
            pltpu.make_async_remote_copy(
                src_ref=ybuf_ref.at[0, 0],
                dst_ref=yret_ref.at[k, e],
                send_sem=ssy.at[0, 0],
                recv_sem=rsy.at[k, e],
                device_id=(me,),
                device_id_type=pl.DeviceIdType.MESH,
            ).wait_recv()


def kernel(x, router_W, route_idx, expert_W, shared_W):
    n_tok = x.shape[0]
    me = lax.axis_index("i")

    scores = jnp.dot(x, router_W, preferred_element_type=jnp.float32)
    probs = jax.nn.softmax(scores, axis=-1)
    p = jnp.take_along_axis(probs, route_idx, axis=1)[:, 0]
    e_glob = route_idx[:, 0]
    dest = e_glob // E_LOC
    local_e = e_glob % E_LOC
    k_off = jnp.remainder(dest - me, N_DEV)
    oh = (e_glob[:, None] == jnp.arange(E_TOT, dtype=e_glob.dtype)).astype(jnp.int32)
    rank = jnp.take_along_axis(jnp.cumsum(oh, axis=0), e_glob[:, None], axis=1)[:, 0] - 1
    valid = rank < C
    slot = (k_off * E_LOC + local_e) * C + rank
    n_slots = N_DEV * E_LOC * C
    slot_safe = jnp.where(valid, slot, n_slots)
    g = jnp.zeros((n_slots,), jnp.int32).at[slot_safe].set(
        jnp.arange(n_tok, dtype=jnp.int32), mode="drop")
    xs = x * p[:, None]
    x_send = xs[g].reshape(N_DEV, E_LOC, C, D)

    out_shared, y_ret = pl.pallas_call(
        _body,
        out_shape=[
            jax.ShapeDtypeStruct((n_tok, D), jnp.float32),
            jax.ShapeDtypeStruct((N_DEV, E_LOC, C, D), jnp.float32),
        ],
        in_specs=[
            pl.BlockSpec(memory_space=pltpu.VMEM),
            pl.BlockSpec(memory_space=pltpu.HBM),
            pl.BlockSpec(memory_space=pltpu.VMEM),
            pl.BlockSpec(memory_space=pltpu.HBM),
        ],
        out_specs=[
            pl.BlockSpec(memory_space=pltpu.VMEM),
            pl.BlockSpec(memory_space=pltpu.HBM),
        ],
        scratch_shapes=[
            pltpu.VMEM((N_DEV, E_LOC, C, D), jnp.float32),
            pltpu.VMEM((2, D, D), jnp.float32),
            pltpu.VMEM((2, N_DEV, C, D), jnp.float32),
            pltpu.SemaphoreType.DMA((N_DEV,)),
            pltpu.SemaphoreType.DMA((N_DEV,)),
            pltpu.SemaphoreType.DMA((2, N_DEV)),
            pltpu.SemaphoreType.DMA((N_DEV, E_LOC)),
            pltpu.SemaphoreType.DMA((2,)),
            pltpu.SemaphoreType.DMA((2,)),
            pltpu.SemaphoreType.DMA((2,)),
        ],
        compiler_params=pltpu.CompilerParams(collective_id=0),
    )(x, x_send, shared_W, expert_W)

    y_flat = y_ret.reshape(n_slots, D)
    contrib = jnp.take(y_flat, jnp.clip(slot, 0, n_slots - 1), axis=0)
    return out_shared + jnp.where(valid[:, None], contrib, 0.0)


# device time: 337407 ns/iter; 1.5346x vs baseline; 1.5346x over previous
import jax
import jax.numpy as jnp
from jax import lax
from jax.experimental import pallas as pl
from jax.experimental.pallas import tpu as pltpu

N_DEV = 8
E_LOC = 8
E_TOT = N_DEV * E_LOC
C = 64
BLK = E_LOC * C
D = 1024
N_TOK = 2048


def _body(x_ref, prow_ref, srow_ref, scol_ref, sw_ref, ew_ref,
          out_ref, yret_ref,
          xrecv_ref, sendbuf_ref, we_ref, ybuf_ref, ystage_ref,
          ssx, rsx, ssy, rsy, wsem, ysem, stsem):
    me = lax.axis_index("i")

    barrier = pltpu.get_barrier_semaphore()
    for k in range(1, N_DEV):
        peer = lax.rem(me + k, N_DEV)
        pl.semaphore_signal(barrier, inc=1, device_id=(peer,),
                            device_id_type=pl.DeviceIdType.MESH)
    pl.semaphore_wait(barrier, N_DEV - 1)

    srow = srow_ref[...]
    prow = prow_ref[...]

    def gather_block(k):
        iota = lax.broadcasted_iota(jnp.int32, (BLK, N_TOK), 0) + (k * BLK)
        gp = jnp.where(iota == srow, prow, 0.0)
        return jnp.dot(gp, x_ref[...], preferred_element_type=jnp.float32)

    disp = {}
    for k in range(1, N_DEV):
        par = k % 2
        if k >= 3:
            disp[k - 2].wait_send()
        sendbuf_ref[par] = gather_block(k).reshape(E_LOC, C, D)
        peer = lax.rem(me + k, N_DEV)
        r = pltpu.make_async_remote_copy(
            src_ref=sendbuf_ref.at[par],
            dst_ref=xrecv_ref.at[k],
            send_sem=ssx.at[k],
            recv_sem=rsx.at[k],
            device_id=(peer,),
            device_id_type=pl.DeviceIdType.MESH,
        )
        r.start()
        disp[k] = r
    pltpu.make_async_copy(ew_ref.at[0], we_ref.at[0], wsem.at[0]).start()
    xrecv_ref[0] = gather_block(0).reshape(E_LOC, C, D)

    out_ref[...] = jnp.dot(x_ref[...], sw_ref[...],
                           preferred_element_type=jnp.float32)

    for k in range(1, N_DEV):
        disp[k].wait_recv()

    ret_rdmas = {0: [], 1: []}
    ret_local = {0: None, 1: None}
    for e in range(E_LOC):
        par = e % 2
        pltpu.make_async_copy(ew_ref.at[e], we_ref.at[par], wsem.at[par]).wait()
        if e + 1 < E_LOC:
            pltpu.make_async_copy(ew_ref.at[e + 1], we_ref.at[(e + 1) % 2],
                                  wsem.at[(e + 1) % 2]).start()
        if e >= 2:
            for r in ret_rdmas[par]:
                r.wait_send()
            ret_local[par].wait()
        a = xrecv_ref[:, e, :, :].reshape(N_DEV * C, D)
        y = jnp.dot(a, we_ref[par], preferred_element_type=jnp.float32)
        ybuf_ref[par] = y.reshape(N_DEV, C, D)
        lst = []
        for k in range(1, N_DEV):
            src_dev = lax.rem(me - k + N_DEV, N_DEV)
            r = pltpu.make_async_remote_copy(
                src_ref=ybuf_ref.at[par, k],
                dst_ref=yret_ref.at[k, e],
                send_sem=ssy.at[par, k],
                recv_sem=rsy.at[k, e],
                device_id=(src_dev,),
                device_id_type=pl.DeviceIdType.MESH,
            )
            r.start()
            lst.append(r)
        ret_rdmas[par] = lst
        lc = pltpu.make_async_copy(ybuf_ref.at[par, 0], yret_ref.at[0, e],
                                   ysem.at[par])
        lc.start()
        ret_local[par] = lc

    for par in (0, 1):
        for r in ret_rdmas[par]:
            r.wait_send()
        ret_local[par].wait()
    for k in (6, 7):
        disp[k].wait_send()

    def wait_block(k):
        if k == 0:
            return
        for e in range(E_LOC):
            pltpu.make_async_remote_copy(
                src_ref=ybuf_ref.at[0, 0],
                dst_ref=yret_ref.at[k, e],
                send_sem=ssy.at[0, 0],
                recv_sem=rsy.at[k, e],
                device_id=(me,),
                device_id_type=pl.DeviceIdType.MESH,
            ).wait_recv()

    scol = scol_ref[...]
    wait_block(0)
    pltpu.make_async_copy(yret_ref.at[0], ystage_ref.at[0], stsem.at[0]).start()
    for k in range(N_DEV):
        par = k % 2
        pltpu.make_async_copy(yret_ref.at[k], ystage_ref.at[par],
                              stsem.at[par]).wait()
        if k + 1 < N_DEV:
            wait_block(k + 1)
            pltpu.make_async_copy(yret_ref.at[k + 1],
                                  ystage_ref.at[(k + 1) % 2],
                                  stsem.at[(k + 1) % 2]).start()
        iota = lax.broadcasted_iota(jnp.int32, (N_TOK, BLK), 1) + (k * BLK)
        s_blk = (iota == scol).astype(jnp.float32)
        y_blk = ystage_ref[par].reshape(BLK, D)
        out_ref[...] = out_ref[...] + jnp.dot(
            s_blk, y_blk, preferred_element_type=jnp.float32)


def kernel(x, router_W, route_idx, expert_W, shared_W):
    n_tok = x.shape[0]
    me = lax.axis_index("i")

    scores = jnp.dot(x, router_W, preferred_element_type=jnp.float32)
    probs = jax.nn.softmax(scores, axis=-1)
    oh = route_idx == jnp.arange(E_TOT, dtype=route_idx.dtype)[None, :]
    p = jnp.sum(probs * oh.astype(jnp.float32), axis=1)
    rank = jnp.sum(jnp.where(oh, jnp.cumsum(oh.astype(jnp.int32), axis=0), 0),
                   axis=1) - 1
    e_glob = route_idx[:, 0]
    dest = e_glob // E_LOC
    local_e = e_glob % E_LOC
    k_off = jnp.remainder(dest - me, N_DEV)
    slot = (k_off * E_LOC + local_e) * C + rank
    slot = jnp.where(rank < C, slot, -1)

    out, _ = pl.pallas_call(
        _body,
        out_shape=[
            jax.ShapeDtypeStruct((n_tok, D), jnp.float32),
            jax.ShapeDtypeStruct((N_DEV, E_LOC, C, D), jnp.float32),
        ],
        in_specs=[
            pl.BlockSpec(memory_space=pltpu.VMEM),
            pl.BlockSpec(memory_space=pltpu.VMEM),
            pl.BlockSpec(memory_space=pltpu.VMEM),
            pl.BlockSpec(memory_space=pltpu.VMEM),
            pl.BlockSpec(memory_space=pltpu.VMEM),
            pl.BlockSpec(memory_space=pltpu.HBM),
        ],
        out_specs=[
            pl.BlockSpec(memory_space=pltpu.VMEM),
            pl.BlockSpec(memory_space=pltpu.HBM),
        ],
        scratch_shapes=[
            pltpu.VMEM((N_DEV, E_LOC, C, D), jnp.float32),
            pltpu.VMEM((2, E_LOC, C, D), jnp.float32),
            pltpu.VMEM((2, D, D), jnp.float32),
            pltpu.VMEM((2, N_DEV, C, D), jnp.float32),
            pltpu.VMEM((2, E_LOC, C, D), jnp.float32),
            pltpu.SemaphoreType.DMA((N_DEV,)),
            pltpu.SemaphoreType.DMA((N_DEV,)),
            pltpu.SemaphoreType.DMA((2, N_DEV)),
            pltpu.SemaphoreType.DMA((N_DEV, E_LOC)),
            pltpu.SemaphoreType.DMA((2,)),
            pltpu.SemaphoreType.DMA((2,)),
            pltpu.SemaphoreType.DMA((2,)),
        ],
        compiler_params=pltpu.CompilerParams(
            collective_id=0, vmem_limit_bytes=64 * 1024 * 1024),
    )(x, p.reshape(1, n_tok), slot.reshape(1, n_tok),
      slot.reshape(n_tok, 1), shared_W, expert_W)
    return out


# device time: 189656 ns/iter; 2.7302x vs baseline; 1.7790x over previous
import jax
import jax.numpy as jnp
from jax import lax
from jax.experimental import pallas as pl
from jax.experimental.pallas import tpu as pltpu

N_DEV = 8
E_LOC = 8
E_TOT = N_DEV * E_LOC
C = 64
BLK = E_LOC * C
D = 1024
N_TOK = 2048


def _body(x_ref, prow_ref, srow_ref, scol_ref, sw_ref, ew_ref,
          out_ref, yret_ref,
          xrecv_ref, sendbuf_ref, we_ref, ybuf_ref, ystage_ref,
          ssx, rsx, ssy, rsy, wsem, ysem, stsem):
    me = lax.axis_index("i")

    barrier = pltpu.get_barrier_semaphore()
    for k in range(1, N_DEV):
        peer = lax.rem(me + k, N_DEV)
        pl.semaphore_signal(barrier, inc=1, device_id=(peer,),
                            device_id_type=pl.DeviceIdType.MESH)
    pl.semaphore_wait(barrier, N_DEV - 1)

    srow = srow_ref[...]
    prow = prow_ref[...]

    def gather_block(k):
        iota = lax.broadcasted_iota(jnp.int32, (BLK, N_TOK), 0) + (k * BLK)
        gp = jnp.where(iota == srow, prow, 0.0)
        out = jnp.dot(gp, x_ref[...], preferred_element_type=jnp.float32)
        return out.astype(jnp.bfloat16)

    disp = {}
    for k in range(1, N_DEV):
        par = k % 4
        if k >= 5:
            disp[k - 4].wait_send()
        sendbuf_ref[par] = gather_block(k).reshape(E_LOC, C, D)
        peer = lax.rem(me + k, N_DEV)
        r = pltpu.make_async_remote_copy(
            src_ref=sendbuf_ref.at[par],
            dst_ref=xrecv_ref.at[k],
            send_sem=ssx.at[k],
            recv_sem=rsx.at[k],
            device_id=(peer,),
            device_id_type=pl.DeviceIdType.MESH,
        )
        r.start()
        disp[k] = r
    pltpu.make_async_copy(ew_ref.at[0], we_ref.at[0], wsem.at[0]).start()
    xrecv_ref[0] = gather_block(0).reshape(E_LOC, C, D)

    out_ref[...] = jnp.dot(x_ref[...], sw_ref[...],
                           preferred_element_type=jnp.float32)

    for k in range(1, N_DEV):
        disp[k].wait_recv()

    ret_rdmas = {0: [], 1: []}
    ret_local = {0: None, 1: None}
    for e in range(E_LOC):
        par = e % 2
        pltpu.make_async_copy(ew_ref.at[e], we_ref.at[par], wsem.at[par]).wait()
        if e + 1 < E_LOC:
            pltpu.make_async_copy(ew_ref.at[e + 1], we_ref.at[(e + 1) % 2],
                                  wsem.at[(e + 1) % 2]).start()
        if e >= 2:
            for r in ret_rdmas[par]:
                r.wait_send()
            ret_local[par].wait()
        a = xrecv_ref[:, e, :, :].reshape(N_DEV * C, D)
        y = jnp.dot(a, we_ref[par].astype(jnp.bfloat16),
                    preferred_element_type=jnp.float32)
        ybuf_ref[par] = y.astype(jnp.bfloat16).reshape(N_DEV, C, D)
        lst = []
        for k in range(1, N_DEV):
            src_dev = lax.rem(me - k + N_DEV, N_DEV)
            r = pltpu.make_async_remote_copy(
                src_ref=ybuf_ref.at[par, k],
                dst_ref=yret_ref.at[k, e],
                send_sem=ssy.at[par, k],
                recv_sem=rsy.at[k, e],
                device_id=(src_dev,),
                device_id_type=pl.DeviceIdType.MESH,
            )
            r.start()
            lst.append(r)
        ret_rdmas[par] = lst
        lc = pltpu.make_async_copy(ybuf_ref.at[par, 0], yret_ref.at[0, e],
                                   ysem.at[par])
        lc.start()
        ret_local[par] = lc

    for par in (0, 1):
        for r in ret_rdmas[par]:
            r.wait_send()
        ret_local[par].wait()
    for k in (4, 5, 6, 7):
        disp[k].wait_send()

    def wait_block(k):
        if k == 0:
            return
        for e in range(E_LOC):
            pltpu.make_async_remote_copy(
                src_ref=ybuf_ref.at[0, 0],
                dst_ref=yret_ref.at[k, e],
                send_sem=ssy.at[0, 0],
                recv_sem=rsy.at[k, e],
                device_id=(me,),
                device_id_type=pl.DeviceIdType.MESH,
            ).wait_recv()

    scol = scol_ref[...]
    wait_block(0)
    pltpu.make_async_copy(yret_ref.at[0], ystage_ref.at[0], stsem.at[0]).start()
    for k in range(N_DEV):
        par = k % 2
        pltpu.make_async_copy(yret_ref.at[k], ystage_ref.at[par],
                              stsem.at[par]).wait()
        if k + 1 < N_DEV:
            wait_block(k + 1)
            pltpu.make_async_copy(yret_ref.at[k + 1],
                                  ystage_ref.at[(k + 1) % 2],
                                  stsem.at[(k + 1) % 2]).start()
        iota = lax.broadcasted_iota(jnp.int32, (N_TOK, BLK), 1) + (k * BLK)
        s_blk = (iota == scol).astype(jnp.bfloat16)
        y_blk = ystage_ref[par].reshape(BLK, D)
        out_ref[...] = out_ref[...] + jnp.dot(
            s_blk, y_blk, preferred_element_type=jnp.float32)


def kernel(x, router_W, route_idx, expert_W, shared_W):
    n_tok = x.shape[0]
    me = lax.axis_index("i")

    scores = jnp.dot(x, router_W, preferred_element_type=jnp.float32)
    probs = jax.nn.softmax(scores, axis=-1)
    oh = route_idx == jnp.arange(E_TOT, dtype=route_idx.dtype)[None, :]
    p = jnp.sum(probs * oh.astype(jnp.float32), axis=1)
    rank = jnp.sum(jnp.where(oh, jnp.cumsum(oh.astype(jnp.int32), axis=0), 0),
                   axis=1) - 1
    e_glob = route_idx[:, 0]
    dest = e_glob // E_LOC
    local_e = e_glob % E_LOC
    k_off = jnp.remainder(dest - me, N_DEV)
    slot = (k_off * E_LOC + local_e) * C + rank
    slot = jnp.where(rank < C, slot, -1)

    out, _ = pl.pallas_call(
        _body,
        out_shape=[
            jax.ShapeDtypeStruct((n_tok, D), jnp.float32),
            jax.ShapeDtypeStruct((N_DEV, E_LOC, C, D), jnp.bfloat16),
        ],
        in_specs=[
            pl.BlockSpec(memory_space=pltpu.VMEM),
            pl.BlockSpec(memory_space=pltpu.VMEM),
            pl.BlockSpec(memory_space=pltpu.VMEM),
            pl.BlockSpec(memory_space=pltpu.VMEM),
            pl.BlockSpec(memory_space=pltpu.VMEM),
            pl.BlockSpec(memory_space=pltpu.HBM),
        ],
        out_specs=[
            pl.BlockSpec(memory_space=pltpu.VMEM),
            pl.BlockSpec(memory_space=pltpu.HBM),
        ],
        scratch_shapes=[
            pltpu.VMEM((N_DEV, E_LOC, C, D), jnp.bfloat16),
            pltpu.VMEM((4, E_LOC, C, D), jnp.bfloat16),
            pltpu.VMEM((2, D, D), jnp.float32),
            pltpu.VMEM((2, N_DEV, C, D), jnp.bfloat16),
            pltpu.VMEM((2, E_LOC, C, D), jnp.bfloat16),
            pltpu.SemaphoreType.DMA((N_DEV,)),
            pltpu.SemaphoreType.DMA((N_DEV,)),
            pltpu.SemaphoreType.DMA((2, N_DEV)),
            pltpu.SemaphoreType.DMA((N_DEV, E_LOC)),
            pltpu.SemaphoreType.DMA((2,)),
            pltpu.SemaphoreType.DMA((2,)),
            pltpu.SemaphoreType.DMA((2,)),
        ],
        compiler_params=pltpu.CompilerParams(
            collective_id=0, vmem_limit_bytes=64 * 1024 * 1024),
    )(x, p.reshape(1, n_tok), slot.reshape(1, n_tok),
      slot.reshape(n_tok, 1), shared_W, expert_W)
    return out


# device time: 175085 ns/iter; 2.9574x vs baseline; 1.0832x over previous
import jax
import jax.numpy as jnp
from jax import lax
from jax.experimental import pallas as pl
from jax.experimental.pallas import tpu as pltpu

N_DEV = 8
E_LOC = 8
E_TOT = N_DEV * E_LOC
C = 64
BLK = E_LOC * C
D = 1024
N_TOK = 2048


def _body(x_ref, prow_ref, srow_ref, scol_ref, sw_ref, ew_ref,
          out_ref,
          xrecv_ref, sendbuf_ref, we_ref, ybuf_ref, yret_ref,
          ssx, rsx, ssy, rsy, wsem):
    me = lax.axis_index("i")

    barrier = pltpu.get_barrier_semaphore()
    for k in range(1, N_DEV):
        peer = lax.rem(me + k, N_DEV)
        pl.semaphore_signal(barrier, inc=1, device_id=(peer,),
                            device_id_type=pl.DeviceIdType.MESH)
    pl.semaphore_wait(barrier, N_DEV - 1)

    srow = srow_ref[...]
    prow = prow_ref[...]

    def gather_block(k):
        iota = lax.broadcasted_iota(jnp.int32, (BLK, N_TOK), 0) + (k * BLK)
        gp = jnp.where(iota == srow, prow, 0.0)
        out = jnp.dot(gp, x_ref[...], preferred_element_type=jnp.float32)
        return out.astype(jnp.bfloat16)

    disp = {}
    for k in range(1, N_DEV):
        par = k % 4
        if k >= 5:
            disp[k - 4].wait_send()
        sendbuf_ref[par] = gather_block(k).reshape(E_LOC, C, D)
        peer = lax.rem(me + k, N_DEV)
        r = pltpu.make_async_remote_copy(
            src_ref=sendbuf_ref.at[par],
            dst_ref=xrecv_ref.at[k],
            send_sem=ssx.at[k],
            recv_sem=rsx.at[k],
            device_id=(peer,),
            device_id_type=pl.DeviceIdType.MESH,
        )
        r.start()
        disp[k] = r
    pltpu.make_async_copy(ew_ref.at[0], we_ref.at[0], wsem.at[0]).start()
    xrecv_ref[0] = gather_block(0).reshape(E_LOC, C, D)

    out_ref[...] = jnp.dot(x_ref[...], sw_ref[...],
                           preferred_element_type=jnp.float32)

    for k in range(1, N_DEV):
        disp[k].wait_recv()

    scol = scol_ref[...]

    def combine_expert(ec):
        for k in range(1, N_DEV):
            pltpu.make_async_remote_copy(
                src_ref=ybuf_ref.at[0, 0],
                dst_ref=yret_ref.at[k, ec],
                send_sem=ssy.at[0, 0],
                recv_sem=rsy.at[k, ec],
                device_id=(me,),
                device_id_type=pl.DeviceIdType.MESH,
            ).wait_recv()
        yb = yret_ref[:, ec, :, :].reshape(N_DEV * C, D)
        iota = lax.broadcasted_iota(jnp.int32, (N_TOK, N_DEV * C), 1)
        tgt = ((iota // C) * E_LOC + ec) * C + jnp.remainder(iota, C)
        s_blk = (tgt == scol).astype(jnp.bfloat16)
        out_ref[...] = out_ref[...] + jnp.dot(
            s_blk, yb, preferred_element_type=jnp.float32)

    ret_rdmas = {0: [], 1: []}
    for e in range(E_LOC):
        par = e % 2
        pltpu.make_async_copy(ew_ref.at[e], we_ref.at[par], wsem.at[par]).wait()
        if e + 1 < E_LOC:
            pltpu.make_async_copy(ew_ref.at[e + 1], we_ref.at[(e + 1) % 2],
                                  wsem.at[(e + 1) % 2]).start()
        if e >= 2:
            for r in ret_rdmas[par]:
                r.wait_send()
        a = xrecv_ref[:, e, :, :].reshape(N_DEV * C, D)
        y = jnp.dot(a, we_ref[par].astype(jnp.bfloat16),
                    preferred_element_type=jnp.float32)
        yb = y.astype(jnp.bfloat16).reshape(N_DEV, C, D)
        yret_ref[0, e] = yb[0]
        ybuf_ref[par] = yb
        lst = []
        for k in range(1, N_DEV):
            src_dev = lax.rem(me - k + N_DEV, N_DEV)
            r = pltpu.make_async_remote_copy(
                src_ref=ybuf_ref.at[par, k],
                dst_ref=yret_ref.at[k, e],
                send_sem=ssy.at[par, k],
                recv_sem=rsy.at[k, e],
                device_id=(src_dev,),
                device_id_type=pl.DeviceIdType.MESH,
            )
            r.start()
            lst.append(r)
        ret_rdmas[par] = lst
        if e >= 1:
            combine_expert(e - 1)
    combine_expert(E_LOC - 1)

    for par in (0, 1):
        for r in ret_rdmas[par]:
            r.wait_send()
    for k in (4, 5, 6, 7):
        disp[k].wait_send()


def kernel(x, router_W, route_idx, expert_W, shared_W):
    n_tok = x.shape[0]
    me = lax.axis_index("i")

    scores = jnp.dot(x, router_W, preferred_element_type=jnp.float32)
    probs = jax.nn.softmax(scores, axis=-1)
    oh = route_idx == jnp.arange(E_TOT, dtype=route_idx.dtype)[None, :]
    p = jnp.sum(probs * oh.astype(jnp.float32), axis=1)
    rank = jnp.sum(jnp.where(oh, jnp.cumsum(oh.astype(jnp.int32), axis=0), 0),
                   axis=1) - 1
    e_glob = route_idx[:, 0]
    dest = e_glob // E_LOC
    local_e = e_glob % E_LOC
    k_off = jnp.remainder(dest - me, N_DEV)
    slot = (k_off * E_LOC + local_e) * C + rank
    slot = jnp.where(rank < C, slot, -1)

    return pl.pallas_call(
        _body,
        out_shape=jax.ShapeDtypeStruct((n_tok, D), jnp.float32),
        in_specs=[
            pl.BlockSpec(memory_space=pltpu.VMEM),
            pl.BlockSpec(memory_space=pltpu.VMEM),
            pl.BlockSpec(memory_space=pltpu.VMEM),
            pl.BlockSpec(memory_space=pltpu.VMEM),
            pl.BlockSpec(memory_space=pltpu.VMEM),
            pl.BlockSpec(memory_space=pltpu.HBM),
        ],
        out_specs=pl.BlockSpec(memory_space=pltpu.VMEM),
        scratch_shapes=[
            pltpu.VMEM((N_DEV, E_LOC, C, D), jnp.bfloat16),
            pltpu.VMEM((4, E_LOC, C, D), jnp.bfloat16),
            pltpu.VMEM((2, D, D), jnp.float32),
            pltpu.VMEM((2, N_DEV, C, D), jnp.bfloat16),
            pltpu.VMEM((N_DEV, E_LOC, C, D), jnp.bfloat16),
            pltpu.SemaphoreType.DMA((N_DEV,)),
            pltpu.SemaphoreType.DMA((N_DEV,)),
            pltpu.SemaphoreType.DMA((2, N_DEV)),
            pltpu.SemaphoreType.DMA((N_DEV, E_LOC)),
            pltpu.SemaphoreType.DMA((2,)),
        ],
        compiler_params=pltpu.CompilerParams(
            collective_id=0, vmem_limit_bytes=64 * 1024 * 1024),
    )(x, p.reshape(1, n_tok), slot.reshape(1, n_tok),
      slot.reshape(n_tok, 1), shared_W, expert_W)
